# baseline (speedup 1.0000x reference)
import jax
import jax.numpy as jnp
from jax.experimental import pallas as pl
from jax.experimental.pallas import tpu as pltpu

S, B, V, D, H, G = 60, 256, 50000, 300, 512, 15
DM, DOUT = 1024, 2
KP = S // G
DP = 384
H4 = 4 * H
BBLK = 16


NCHUNK = S * B // 256
VPAD = 512


def _embed_kernel(toks_ref, tbl_hbm, wp_ref, pb_ref, y_ref, st_ref,
                  tbl_vmem, tile, sem):
    p = pl.program_id(0)
    c = pl.program_id(1)

    @pl.when(c == 0)
    def _():
        cp = pltpu.make_async_copy(tbl_hbm, tbl_vmem, sem)
        cp.start()
        cp.wait()

    off = p * (S * B) + c * 256
    for mi in range(256):
        t = pl.multiple_of(toks_ref[off + mi], 2)
        tile[2 * mi:2 * mi + 2, :] = tbl_vmem[pl.ds(t, 2), :]

    tv = pltpu.bitcast(tile[...], jnp.bfloat16)
    e4 = tv.reshape(256, 4, 128)
    y = pb_ref[...].astype(jnp.float32)
    acc = None
    for c4 in range(4):
        d = jnp.dot(e4[:, c4, :], wp_ref[128 * c4:128 * (c4 + 1), :],
                    preferred_element_type=jnp.float32)
        acc = d if acc is None else acc + d
    y = acc + y
    y_ref[0] = y

    @pl.when(c == 0)
    def _():
        st_ref[...] = jnp.zeros_like(st_ref)

    st_ref[0, 0:1, :] = st_ref[0, 0:1, :] + jnp.sum(y, axis=0, keepdims=True)
    st_ref[0, 1:2, :] = st_ref[0, 1:2, :] + jnp.sum(y * y, axis=0,
                                                    keepdims=True)


def _run_embed(toks, tbl_i32, wproj, pbias):
    return pl.pallas_call(
        _embed_kernel,
        out_shape=(jax.ShapeDtypeStruct((2, S * B, DP), jnp.float32),
                   jax.ShapeDtypeStruct((2, 2, DP), jnp.float32)),
        grid_spec=pltpu.PrefetchScalarGridSpec(
            num_scalar_prefetch=1,
            grid=(2, NCHUNK),
            in_specs=[
                pl.BlockSpec(memory_space=pl.ANY),
                pl.BlockSpec((VPAD, DP), lambda p, c, toks: (0, 0)),
                pl.BlockSpec((1, DP), lambda p, c, toks: (0, 0)),
            ],
            out_specs=[
                pl.BlockSpec((1, 256, DP), lambda p, c, toks: (p, c, 0)),
                pl.BlockSpec((1, 2, DP), lambda p, c, toks: (p, 0, 0)),
            ],
            scratch_shapes=[
                pltpu.VMEM((2 * V, 128), jnp.int32),
                pltpu.VMEM((512, 128), jnp.int32),
                pltpu.SemaphoreType.DMA,
            ],
        ),
        compiler_params=pltpu.CompilerParams(
            dimension_semantics=("parallel", "arbitrary"),
            vmem_limit_bytes=56 * 1024 * 1024),
        name="embed_gather_proj",
    )(toks, tbl_i32, wproj, pbias)


def _sig(x):
    return 1.0 / (1.0 + jnp.exp(-x))


def _tanh(x):
    e = jnp.exp(-2.0 * x)
    return (1.0 - e) / (1.0 + e)


def _hilo(x):
    hi = x.astype(jnp.bfloat16)
    lo = (x - hi.astype(jnp.float32)).astype(jnp.bfloat16)
    return hi, lo


def _dot3(xhi, xlo, whi, wlo):
    return (jnp.dot(xhi, whi, preferred_element_type=jnp.float32)
            + jnp.dot(xhi, wlo, preferred_element_type=jnp.float32)
            + jnp.dot(xlo, whi, preferred_element_type=jnp.float32))


def _lstm_kernel(y_hbm, st_ref, bng_ref, bnb_ref,
                 wih_ref, whh_ref, bi_ref, bh_ref, hs_hbm,
                 xbuf, h_ref, c_ref, hout, sem_in, sem_out):
    p = pl.program_id(0)
    bsum = bi_ref[...] + bh_ref[...]
    n = jnp.float32(S * B)
    mu = st_ref[0, 0:1, :] / n
    var = st_ref[0, 1:2, :] / n - mu * mu
    rs = jax.lax.rsqrt(var + 1e-5)
    bng = bng_ref[...]
    bnb = bnb_ref[...]
    h_ref[...] = jnp.zeros_like(h_ref)
    c_ref[...] = jnp.zeros_like(c_ref)
    wih = wih_ref[...]
    whh = whh_ref[...]

    pltpu.make_async_copy(y_hbm.at[p, pl.ds(0, B)], xbuf.at[0],
                          sem_in.at[0]).start()

    def step(s, carry):
        slot = jax.lax.rem(s, 2)
        nslot = jax.lax.rem(s + 1, 2)

        @pl.when(s + 1 < S)
        def _():
            pltpu.make_async_copy(y_hbm.at[p, pl.ds((s + 1) * B, B)],
                                  xbuf.at[nslot], sem_in.at[nslot]).start()

        pltpu.make_async_copy(y_hbm.at[p, pl.ds(s * B, B)], xbuf.at[slot],
                              sem_in.at[slot]).wait()
        x = ((((xbuf[slot] - mu) * rs) * bng) + bnb).astype(jnp.bfloat16)
        hb = h_ref[...].astype(jnp.bfloat16)
        z = (jnp.dot(x, wih, preferred_element_type=jnp.float32)
             + jnp.dot(hb, whh, preferred_element_type=jnp.float32)
             + bsum)
        zi = z[:, 0 * H:1 * H]
        zf = z[:, 1 * H:2 * H]
        zg = z[:, 2 * H:3 * H]
        zo = z[:, 3 * H:4 * H]
        c = _sig(zf) * c_ref[...] + _sig(zi) * _tanh(zg)
        h = _sig(zo) * _tanh(c)
        c_ref[...] = c
        h_ref[...] = h

        @pl.when(s >= 2)
        def _():
            pltpu.make_async_copy(hout.at[slot], hs_hbm.at[p, s - 2],
                                  sem_out.at[slot]).wait()

        hout[slot] = h
        pltpu.make_async_copy(hout.at[slot], hs_hbm.at[p, s],
                              sem_out.at[slot]).start()
        return carry

    jax.lax.fori_loop(0, S, step, 0)
    pltpu.make_async_copy(hout.at[S % 2], hs_hbm.at[p, S - 2],
                          sem_out.at[S % 2]).wait()
    pltpu.make_async_copy(hout.at[(S - 1) % 2], hs_hbm.at[p, S - 1],
                          sem_out.at[(S - 1) % 2]).wait()


def _run_lstm(y_all, stats, bng, bnb, wih_t, whh_t, bih, bhh):
    return pl.pallas_call(
        _lstm_kernel,
        out_shape=jax.ShapeDtypeStruct((2, S, B, H), jnp.float32),
        grid=(2,),
        in_specs=[
            pl.BlockSpec(memory_space=pl.ANY),
            pl.BlockSpec((1, 2, DP), lambda p: (p, 0, 0)),
            pl.BlockSpec((1, DP), lambda p: (0, 0)),
            pl.BlockSpec((1, DP), lambda p: (0, 0)),
            pl.BlockSpec((DP, H4), lambda p: (0, 0)),
            pl.BlockSpec((H, H4), lambda p: (0, 0)),
            pl.BlockSpec((1, H4), lambda p: (0, 0)),
            pl.BlockSpec((1, H4), lambda p: (0, 0)),
        ],
        out_specs=pl.BlockSpec(memory_space=pl.ANY),
        scratch_shapes=[
            pltpu.VMEM((2, B, DP), jnp.float32),
            pltpu.VMEM((B, H), jnp.float32),
            pltpu.VMEM((B, H), jnp.float32),
            pltpu.VMEM((2, B, H), jnp.float32),
            pltpu.SemaphoreType.DMA((2,)),
            pltpu.SemaphoreType.DMA((2,)),
        ],
        compiler_params=pltpu.CompilerParams(
            dimension_semantics=("parallel",),
            vmem_limit_bytes=56 * 1024 * 1024),
        name="lstm_encoder",
    )(y_all, stats, bng, bnb, wih_t, whh_t, bih, bhh)


def _dist_kernel(h1_ref, h2_ref, o_ref):
    jj = jax.lax.broadcasted_iota(jnp.int32, (S, G), 1)
    j = jax.lax.broadcasted_iota(jnp.int32, (S, G), 0)
    colsel = (j == KP * jj).astype(jnp.bfloat16)
    ii = jax.lax.broadcasted_iota(jnp.int32, (G, S), 0)
    i2 = jax.lax.broadcasted_iota(jnp.int32, (G, S), 1)
    rowsel = (i2 == KP * ii).astype(jnp.bfloat16)
    ones_row = jnp.ones((1, H), jnp.bfloat16)

    for bi in range(BBLK):
        h1 = h1_ref[0, :, bi * H:(bi + 1) * H]
        h2 = h2_ref[0, :, bi * H:(bi + 1) * H]
        dg = lambda a, b: jax.lax.dot_general(
            a, b, (((1,), (1,)), ((), ())),
            preferred_element_type=jnp.float32)
        g = dg(h1.astype(jnp.bfloat16), h2.astype(jnp.bfloat16))
        n1 = jnp.sum(h1 * h1, axis=1, keepdims=True)
        sqhi, sqlo = _hilo(h2 * h2)
        n2 = dg(ones_row, sqhi) + dg(ones_row, sqlo)
        sq = n1 + n2 - 2.0 * g
        m = sq
        for k in (1, 2, 3):
            m = jnp.minimum(m, jnp.concatenate([sq[:, k:], sq[:, :k]], axis=1))
        mhi, mlo = _hilo(m)
        mc = (jnp.dot(mhi, colsel, preferred_element_type=jnp.float32)
              + jnp.dot(mlo, colsel, preferred_element_type=jnp.float32))
        m2 = mc
        for k in (1, 2, 3):
            m2 = jnp.minimum(m2, jnp.concatenate([mc[k:], mc[:k]], axis=0))
        m2hi, m2lo = _hilo(m2)
        pooled_sq = (jnp.dot(rowsel, m2hi, preferred_element_type=jnp.float32)
                     + jnp.dot(rowsel, m2lo, preferred_element_type=jnp.float32))
        o_ref[bi] = jnp.sqrt(jnp.maximum(pooled_sq, 0.0) + 1e-12)


def _run_dist(hs_flat):
    return pl.pallas_call(
        _dist_kernel,
        out_shape=jax.ShapeDtypeStruct((B, G, G), jnp.float32),
        grid=(B // BBLK,),
        in_specs=[
            pl.BlockSpec((1, S, BBLK * H), lambda i: (0, 0, i)),
            pl.BlockSpec((1, S, BBLK * H), lambda i: (1, 0, i)),
        ],
        out_specs=pl.BlockSpec((BBLK, G, G), lambda i: (i, 0, 0)),
        compiler_params=pltpu.CompilerParams(
            dimension_semantics=("parallel",),
            vmem_limit_bytes=56 * 1024 * 1024),
        name="pair_dist_pool",
    )(hs_flat, hs_flat)


def _bn_cols(x, g, b):
    mu = x.mean(axis=0, keepdims=True)
    var = x.var(axis=0, keepdims=True)
    return (x - mu) * jax.lax.rsqrt(var + 1e-5) * g + b


def _mlp_kernel(x_ref, w1_ref, b1_ref, g1_ref, be1_ref,
                w2_ref, b2_ref, g2_ref, be2_ref, w3_ref, b3_ref, o_ref):
    x = x_ref[...]
    z1 = jnp.dot(x, w1_ref[...], preferred_element_type=jnp.float32) + b1_ref[...]
    x1 = _bn_cols(jnp.maximum(z1, 0.0), g1_ref[...], be1_ref[...])
    z2 = jnp.dot(x1, w2_ref[...], preferred_element_type=jnp.float32) + b2_ref[...]
    x2 = _bn_cols(jnp.maximum(z2, 0.0), g2_ref[...], be2_ref[...])
    z3 = jnp.dot(x2, w3_ref[...], preferred_element_type=jnp.float32) + b3_ref[...]
    lane = jax.lax.broadcasted_iota(jnp.int32, z3.shape, 1)
    zm = jnp.where(lane < DOUT, z3, -jnp.inf)
    mx = jnp.max(zm, axis=-1, keepdims=True)
    lse = jnp.log(jnp.sum(jnp.where(lane < DOUT, jnp.exp(zm - mx), 0.0),
                          axis=-1, keepdims=True)) + mx
    o_ref[...] = z3 - lse


def _run_mlp(x, W1, b1, g1, be1, W2, b2, g2, be2, W3, b3):
    return pl.pallas_call(
        _mlp_kernel,
        out_shape=jax.ShapeDtypeStruct((B, 128), jnp.float32),
        name="mlp_head",
        compiler_params=pltpu.CompilerParams(
            vmem_limit_bytes=56 * 1024 * 1024),
    )(x, W1, b1, g1, be1, W2, b2, g2, be2, W3, b3)


def kernel(embed_table, proj_W, proj_b, bn_g, bn_b, Wih, Whh, bih, bhh,
           W1, b1, g1, be1, W2, b2, g2, be2, W3, b3,
           extra_feats, sentence1, sentence2):
    tblp = jnp.pad(embed_table.astype(jnp.bfloat16),
                   ((0, 0), (0, VPAD - D)))
    tbl_i32 = jax.lax.bitcast_convert_type(
        tblp.reshape(V, VPAD // 2, 2), jnp.int32).reshape(2 * V, 128)
    toks = jnp.concatenate([sentence1.reshape(-1),
                            sentence2.reshape(-1)]).astype(jnp.int32) * 2
    wproj = jnp.pad(proj_W.T, ((0, VPAD - D), (0, DP - D))
                    ).astype(jnp.bfloat16)
    rr = jnp.arange(VPAD)
    cc, ll = rr // 128, rr % 128
    wproj = wproj[256 * (cc // 2) + (cc % 2) + 2 * ll, :]
    pbias = jnp.pad(proj_b, (0, DP - D))[None, :]
    y_all, stats = _run_embed(toks, tbl_i32, wproj, pbias)

    wih_t = jnp.pad(Wih.T, ((0, DP - D), (0, 0))).astype(jnp.bfloat16)
    whh_t = Whh.T.astype(jnp.bfloat16)
    bng = jnp.pad(bn_g, (0, DP - D))[None, :]
    bnb = jnp.pad(bn_b, (0, DP - D))[None, :]
    hs = _run_lstm(y_all, stats, bng, bnb, wih_t, whh_t,
                   bih[None, :], bhh[None, :])

    pooled = _run_dist(hs.reshape(2, S, B * H))

    x = jnp.concatenate([pooled.reshape(B, G * G), extra_feats,
                         jnp.zeros((B, 256 - (G * G + 7)), jnp.float32)],
                        axis=1)
    W1p = jnp.pad(W1.T, ((0, 256 - (G * G + 7)), (0, 0)))
    W3p = jnp.pad(W3.T, ((0, 0), (0, 128 - DOUT)))
    out = _run_mlp(x, W1p, b1[None, :], g1[None, :], be1[None, :],
                   W2.T, b2[None, :], g2[None, :], be2[None, :], W3p,
                   jnp.pad(b3, (0, 126))[None, :])
    return out[:, :DOUT]


# speedup vs baseline: 1.5952x; 1.5952x over previous
import jax
import jax.numpy as jnp
from jax.experimental import pallas as pl
from jax.experimental.pallas import tpu as pltpu

S, B, V, D, H, G = 60, 256, 50000, 300, 512, 15
DM, DOUT = 1024, 2
KP = S // G
DP = 384
H4 = 4 * H
BBLK = 16


NCHUNK = S * B // 256
VPAD = 512


VROWS = 2000


def _pack_kernel(x_ref, o_ref):
    x = x_ref[...]
    a1 = jnp.concatenate(
        [x[:, 256:D], jnp.zeros((VROWS, 128 - (D - 256)), jnp.float32)],
        axis=1)
    zero = jnp.zeros((VROWS, 128), jnp.float32)
    w0 = pltpu.pack_elementwise([x[:, 0:128], x[:, 128:256]],
                                packed_dtype=jnp.bfloat16)
    w1 = pltpu.pack_elementwise([a1, zero], packed_dtype=jnp.bfloat16)
    o_ref[:, 0, :] = w0
    o_ref[:, 1, :] = w1


def _run_pack(embed_table):
    return pl.pallas_call(
        _pack_kernel,
        out_shape=jax.ShapeDtypeStruct((V, 2, 128), jnp.uint32),
        grid=(V // VROWS,),
        in_specs=[pl.BlockSpec((VROWS, D), lambda i: (i, 0))],
        out_specs=pl.BlockSpec((VROWS, 2, 128), lambda i: (i, 0, 0)),
        compiler_params=pltpu.CompilerParams(
            dimension_semantics=("arbitrary",),
            vmem_limit_bytes=56 * 1024 * 1024),
        name="table_pack",
    )(embed_table)


def _embed_kernel(toks_ref, tbl_hbm, wp_ref, pb_ref, y_ref, st_ref,
                  tbl_vmem, tile, sem):
    p = pl.program_id(0)
    c = pl.program_id(1)

    @pl.when(c == 0)
    def _():
        cp = pltpu.make_async_copy(tbl_hbm, tbl_vmem, sem)
        cp.start()
        cp.wait()

    off = p * (S * B) + c * 256
    for mi in range(256):
        t = pl.multiple_of(toks_ref[off + mi], 2)
        tile[2 * mi:2 * mi + 2, :] = tbl_vmem[pl.ds(t, 2), :]

    tv = pltpu.bitcast(tile[...], jnp.bfloat16)
    e4 = tv.reshape(256, 4, 128)
    y = pb_ref[...].astype(jnp.float32)
    acc = None
    for c4 in range(4):
        d = jnp.dot(e4[:, c4, :], wp_ref[128 * c4:128 * (c4 + 1), :],
                    preferred_element_type=jnp.float32)
        acc = d if acc is None else acc + d
    y = acc + y
    y_ref[0] = y

    @pl.when(c == 0)
    def _():
        st_ref[...] = jnp.zeros_like(st_ref)

    st_ref[0, 0:1, :] = st_ref[0, 0:1, :] + jnp.sum(y, axis=0, keepdims=True)
    st_ref[0, 1:2, :] = st_ref[0, 1:2, :] + jnp.sum(y * y, axis=0,
                                                    keepdims=True)


def _run_embed(toks, tbl_i32, wproj, pbias):
    return pl.pallas_call(
        _embed_kernel,
        out_shape=(jax.ShapeDtypeStruct((2, S * B, DP), jnp.float32),
                   jax.ShapeDtypeStruct((2, 2, DP), jnp.float32)),
        grid_spec=pltpu.PrefetchScalarGridSpec(
            num_scalar_prefetch=1,
            grid=(2, NCHUNK),
            in_specs=[
                pl.BlockSpec(memory_space=pl.ANY),
                pl.BlockSpec((VPAD, DP), lambda p, c, toks: (0, 0)),
                pl.BlockSpec((1, DP), lambda p, c, toks: (0, 0)),
            ],
            out_specs=[
                pl.BlockSpec((1, 256, DP), lambda p, c, toks: (p, c, 0)),
                pl.BlockSpec((1, 2, DP), lambda p, c, toks: (p, 0, 0)),
            ],
            scratch_shapes=[
                pltpu.VMEM((2 * V, 128), jnp.int32),
                pltpu.VMEM((512, 128), jnp.int32),
                pltpu.SemaphoreType.DMA,
            ],
        ),
        compiler_params=pltpu.CompilerParams(
            dimension_semantics=("parallel", "arbitrary"),
            vmem_limit_bytes=56 * 1024 * 1024),
        name="embed_gather_proj",
    )(toks, tbl_i32, wproj, pbias)


def _sig(x):
    return 1.0 / (1.0 + jnp.exp(-x))


def _tanh(x):
    e = jnp.exp(-2.0 * x)
    return (1.0 - e) / (1.0 + e)


def _hilo(x):
    hi = x.astype(jnp.bfloat16)
    lo = (x - hi.astype(jnp.float32)).astype(jnp.bfloat16)
    return hi, lo


def _dot3(xhi, xlo, whi, wlo):
    return (jnp.dot(xhi, whi, preferred_element_type=jnp.float32)
            + jnp.dot(xhi, wlo, preferred_element_type=jnp.float32)
            + jnp.dot(xlo, whi, preferred_element_type=jnp.float32))


def _lstm_kernel(y_hbm, st_ref, bng_ref, bnb_ref,
                 wih_ref, whh_ref, bi_ref, bh_ref, hs_hbm,
                 xbuf, h_ref, c_ref, hout, sem_in, sem_out):
    p = pl.program_id(0)
    bsum = bi_ref[...] + bh_ref[...]
    n = jnp.float32(S * B)
    mu = st_ref[0, 0:1, :] / n
    var = st_ref[0, 1:2, :] / n - mu * mu
    rs = jax.lax.rsqrt(var + 1e-5)
    bng = bng_ref[...]
    bnb = bnb_ref[...]
    h_ref[...] = jnp.zeros_like(h_ref)
    c_ref[...] = jnp.zeros_like(c_ref)
    wih = wih_ref[...]
    whh = whh_ref[...]

    pltpu.make_async_copy(y_hbm.at[p, pl.ds(0, B)], xbuf.at[0],
                          sem_in.at[0]).start()

    def step(s, carry):
        slot = jax.lax.rem(s, 2)
        nslot = jax.lax.rem(s + 1, 2)

        @pl.when(s + 1 < S)
        def _():
            pltpu.make_async_copy(y_hbm.at[p, pl.ds((s + 1) * B, B)],
                                  xbuf.at[nslot], sem_in.at[nslot]).start()

        pltpu.make_async_copy(y_hbm.at[p, pl.ds(s * B, B)], xbuf.at[slot],
                              sem_in.at[slot]).wait()
        x = ((((xbuf[slot] - mu) * rs) * bng) + bnb).astype(jnp.bfloat16)
        hb = h_ref[...].astype(jnp.bfloat16)
        z = (jnp.dot(x, wih, preferred_element_type=jnp.float32)
             + jnp.dot(hb, whh, preferred_element_type=jnp.float32)
             + bsum)
        zi = z[:, 0 * H:1 * H]
        zf = z[:, 1 * H:2 * H]
        zg = z[:, 2 * H:3 * H]
        zo = z[:, 3 * H:4 * H]
        c = _sig(zf) * c_ref[...] + _sig(zi) * _tanh(zg)
        h = _sig(zo) * _tanh(c)
        c_ref[...] = c
        h_ref[...] = h

        @pl.when(s >= 2)
        def _():
            pltpu.make_async_copy(hout.at[slot], hs_hbm.at[p, s - 2],
                                  sem_out.at[slot]).wait()

        hout[slot] = h
        pltpu.make_async_copy(hout.at[slot], hs_hbm.at[p, s],
                              sem_out.at[slot]).start()
        return carry

    jax.lax.fori_loop(0, S, step, 0)
    pltpu.make_async_copy(hout.at[S % 2], hs_hbm.at[p, S - 2],
                          sem_out.at[S % 2]).wait()
    pltpu.make_async_copy(hout.at[(S - 1) % 2], hs_hbm.at[p, S - 1],
                          sem_out.at[(S - 1) % 2]).wait()


def _run_lstm(y_all, stats, bng, bnb, wih_t, whh_t, bih, bhh):
    return pl.pallas_call(
        _lstm_kernel,
        out_shape=jax.ShapeDtypeStruct((2, S, B, H), jnp.float32),
        grid=(2,),
        in_specs=[
            pl.BlockSpec(memory_space=pl.ANY),
            pl.BlockSpec((1, 2, DP), lambda p: (p, 0, 0)),
            pl.BlockSpec((1, DP), lambda p: (0, 0)),
            pl.BlockSpec((1, DP), lambda p: (0, 0)),
            pl.BlockSpec((DP, H4), lambda p: (0, 0)),
            pl.BlockSpec((H, H4), lambda p: (0, 0)),
            pl.BlockSpec((1, H4), lambda p: (0, 0)),
            pl.BlockSpec((1, H4), lambda p: (0, 0)),
        ],
        out_specs=pl.BlockSpec(memory_space=pl.ANY),
        scratch_shapes=[
            pltpu.VMEM((2, B, DP), jnp.float32),
            pltpu.VMEM((B, H), jnp.float32),
            pltpu.VMEM((B, H), jnp.float32),
            pltpu.VMEM((2, B, H), jnp.float32),
            pltpu.SemaphoreType.DMA((2,)),
            pltpu.SemaphoreType.DMA((2,)),
        ],
        compiler_params=pltpu.CompilerParams(
            dimension_semantics=("parallel",),
            vmem_limit_bytes=56 * 1024 * 1024),
        name="lstm_encoder",
    )(y_all, stats, bng, bnb, wih_t, whh_t, bih, bhh)


def _dist_kernel(h1_ref, h2_ref, o_ref):
    jj = jax.lax.broadcasted_iota(jnp.int32, (S, G), 1)
    j = jax.lax.broadcasted_iota(jnp.int32, (S, G), 0)
    colsel = (j == KP * jj).astype(jnp.bfloat16)
    ii = jax.lax.broadcasted_iota(jnp.int32, (G, S), 0)
    i2 = jax.lax.broadcasted_iota(jnp.int32, (G, S), 1)
    rowsel = (i2 == KP * ii).astype(jnp.bfloat16)
    ones_row = jnp.ones((1, H), jnp.bfloat16)

    for bi in range(BBLK):
        h1 = h1_ref[0, :, bi * H:(bi + 1) * H]
        h2 = h2_ref[0, :, bi * H:(bi + 1) * H]
        dg = lambda a, b: jax.lax.dot_general(
            a, b, (((1,), (1,)), ((), ())),
            preferred_element_type=jnp.float32)
        g = dg(h1.astype(jnp.bfloat16), h2.astype(jnp.bfloat16))
        n1 = jnp.sum(h1 * h1, axis=1, keepdims=True)
        sqhi, sqlo = _hilo(h2 * h2)
        n2 = dg(ones_row, sqhi) + dg(ones_row, sqlo)
        sq = n1 + n2 - 2.0 * g
        m = sq
        for k in (1, 2, 3):
            m = jnp.minimum(m, jnp.concatenate([sq[:, k:], sq[:, :k]], axis=1))
        mhi, mlo = _hilo(m)
        mc = (jnp.dot(mhi, colsel, preferred_element_type=jnp.float32)
              + jnp.dot(mlo, colsel, preferred_element_type=jnp.float32))
        m2 = mc
        for k in (1, 2, 3):
            m2 = jnp.minimum(m2, jnp.concatenate([mc[k:], mc[:k]], axis=0))
        m2hi, m2lo = _hilo(m2)
        pooled_sq = (jnp.dot(rowsel, m2hi, preferred_element_type=jnp.float32)
                     + jnp.dot(rowsel, m2lo, preferred_element_type=jnp.float32))
        o_ref[bi] = jnp.sqrt(jnp.maximum(pooled_sq, 0.0) + 1e-12)


def _run_dist(hs_flat):
    return pl.pallas_call(
        _dist_kernel,
        out_shape=jax.ShapeDtypeStruct((B, G, G), jnp.float32),
        grid=(B // BBLK,),
        in_specs=[
            pl.BlockSpec((1, S, BBLK * H), lambda i: (0, 0, i)),
            pl.BlockSpec((1, S, BBLK * H), lambda i: (1, 0, i)),
        ],
        out_specs=pl.BlockSpec((BBLK, G, G), lambda i: (i, 0, 0)),
        compiler_params=pltpu.CompilerParams(
            dimension_semantics=("parallel",),
            vmem_limit_bytes=56 * 1024 * 1024),
        name="pair_dist_pool",
    )(hs_flat, hs_flat)


def _bn_cols(x, g, b):
    mu = x.mean(axis=0, keepdims=True)
    var = x.var(axis=0, keepdims=True)
    return (x - mu) * jax.lax.rsqrt(var + 1e-5) * g + b


def _mlp_kernel(x_ref, w1_ref, b1_ref, g1_ref, be1_ref,
                w2_ref, b2_ref, g2_ref, be2_ref, w3_ref, b3_ref, o_ref):
    x = x_ref[...]
    z1 = jnp.dot(x, w1_ref[...], preferred_element_type=jnp.float32) + b1_ref[...]
    x1 = _bn_cols(jnp.maximum(z1, 0.0), g1_ref[...], be1_ref[...])
    z2 = jnp.dot(x1, w2_ref[...], preferred_element_type=jnp.float32) + b2_ref[...]
    x2 = _bn_cols(jnp.maximum(z2, 0.0), g2_ref[...], be2_ref[...])
    z3 = jnp.dot(x2, w3_ref[...], preferred_element_type=jnp.float32) + b3_ref[...]
    lane = jax.lax.broadcasted_iota(jnp.int32, z3.shape, 1)
    zm = jnp.where(lane < DOUT, z3, -jnp.inf)
    mx = jnp.max(zm, axis=-1, keepdims=True)
    lse = jnp.log(jnp.sum(jnp.where(lane < DOUT, jnp.exp(zm - mx), 0.0),
                          axis=-1, keepdims=True)) + mx
    o_ref[...] = z3 - lse


def _run_mlp(x, W1, b1, g1, be1, W2, b2, g2, be2, W3, b3):
    return pl.pallas_call(
        _mlp_kernel,
        out_shape=jax.ShapeDtypeStruct((B, 128), jnp.float32),
        name="mlp_head",
        compiler_params=pltpu.CompilerParams(
            vmem_limit_bytes=56 * 1024 * 1024),
    )(x, W1, b1, g1, be1, W2, b2, g2, be2, W3, b3)


def kernel(embed_table, proj_W, proj_b, bn_g, bn_b, Wih, Whh, bih, bhh,
           W1, b1, g1, be1, W2, b2, g2, be2, W3, b3,
           extra_feats, sentence1, sentence2):
    tbl_i32 = jax.lax.bitcast_convert_type(
        _run_pack(embed_table), jnp.int32).reshape(2 * V, 128)
    toks = jnp.concatenate([sentence1.reshape(-1),
                            sentence2.reshape(-1)]).astype(jnp.int32) * 2
    wproj = jnp.pad(proj_W.T, ((0, VPAD - D), (0, DP - D))
                    ).astype(jnp.bfloat16)
    pbias = jnp.pad(proj_b, (0, DP - D))[None, :]
    y_all, stats = _run_embed(toks, tbl_i32, wproj, pbias)

    wih_t = jnp.pad(Wih.T, ((0, DP - D), (0, 0))).astype(jnp.bfloat16)
    whh_t = Whh.T.astype(jnp.bfloat16)
    bng = jnp.pad(bn_g, (0, DP - D))[None, :]
    bnb = jnp.pad(bn_b, (0, DP - D))[None, :]
    hs = _run_lstm(y_all, stats, bng, bnb, wih_t, whh_t,
                   bih[None, :], bhh[None, :])

    pooled = _run_dist(hs.reshape(2, S, B * H))

    x = jnp.concatenate([pooled.reshape(B, G * G), extra_feats,
                         jnp.zeros((B, 256 - (G * G + 7)), jnp.float32)],
                        axis=1)
    W1p = jnp.pad(W1.T, ((0, 256 - (G * G + 7)), (0, 0)))
    W3p = jnp.pad(W3.T, ((0, 0), (0, 128 - DOUT)))
    out = _run_mlp(x, W1p, b1[None, :], g1[None, :], be1[None, :],
                   W2.T, b2[None, :], g2[None, :], be2[None, :], W3p,
                   jnp.pad(b3, (0, 126))[None, :])
    return out[:, :DOUT]


# speedup vs baseline: 1.6054x; 1.0064x over previous
import jax
import jax.numpy as jnp
from jax.experimental import pallas as pl
from jax.experimental.pallas import tpu as pltpu

S, B, V, D, H, G = 60, 256, 50000, 300, 512, 15
DM, DOUT = 1024, 2
KP = S // G
DP = 384
H4 = 4 * H
BBLK = 16


NCHUNK = S * B // 256
VPAD = 512


VROWS = 5000


def _pack_kernel(x_ref, o_ref):
    x = x_ref[...]
    a1 = jnp.concatenate(
        [x[:, 256:D], jnp.zeros((VROWS, 128 - (D - 256)), jnp.float32)],
        axis=1)
    zero = jnp.zeros((VROWS, 128), jnp.float32)
    w0 = pltpu.pack_elementwise([x[:, 0:128], x[:, 128:256]],
                                packed_dtype=jnp.bfloat16)
    w1 = pltpu.pack_elementwise([a1, zero], packed_dtype=jnp.bfloat16)
    o_ref[:, 0, :] = w0
    o_ref[:, 1, :] = w1


def _run_pack(embed_table):
    return pl.pallas_call(
        _pack_kernel,
        out_shape=jax.ShapeDtypeStruct((V, 2, 128), jnp.uint32),
        grid=(V // VROWS,),
        in_specs=[pl.BlockSpec((VROWS, D), lambda i: (i, 0))],
        out_specs=pl.BlockSpec((VROWS, 2, 128), lambda i: (i, 0, 0)),
        compiler_params=pltpu.CompilerParams(
            dimension_semantics=("arbitrary",),
            vmem_limit_bytes=56 * 1024 * 1024),
        name="table_pack",
    )(embed_table)


def _embed_kernel(toks_ref, tbl_hbm, wp_ref, pb_ref, y_ref, st_ref,
                  tbl_vmem, tile, sem):
    p = pl.program_id(0)
    c = pl.program_id(1)

    @pl.when(c == 0)
    def _():
        cp = pltpu.make_async_copy(tbl_hbm, tbl_vmem, sem)
        cp.start()
        cp.wait()

    off = p * (S * B) + c * 256
    for mi in range(256):
        t = pl.multiple_of(toks_ref[off + mi], 2)
        tile[2 * mi:2 * mi + 2, :] = tbl_vmem[pl.ds(t, 2), :]

    tv = pltpu.bitcast(tile[...], jnp.bfloat16)
    e4 = tv.reshape(256, 4, 128)
    y = pb_ref[...].astype(jnp.float32)
    acc = None
    for c4 in range(4):
        d = jnp.dot(e4[:, c4, :], wp_ref[128 * c4:128 * (c4 + 1), :],
                    preferred_element_type=jnp.float32)
        acc = d if acc is None else acc + d
    y = acc + y
    y_ref[0] = y

    @pl.when(c == 0)
    def _():
        st_ref[...] = jnp.zeros_like(st_ref)

    st_ref[0, 0:1, :] = st_ref[0, 0:1, :] + jnp.sum(y, axis=0, keepdims=True)
    st_ref[0, 1:2, :] = st_ref[0, 1:2, :] + jnp.sum(y * y, axis=0,
                                                    keepdims=True)


def _run_embed(toks, tbl_i32, wproj, pbias):
    return pl.pallas_call(
        _embed_kernel,
        out_shape=(jax.ShapeDtypeStruct((2, S * B, DP), jnp.float32),
                   jax.ShapeDtypeStruct((2, 2, DP), jnp.float32)),
        grid_spec=pltpu.PrefetchScalarGridSpec(
            num_scalar_prefetch=1,
            grid=(2, NCHUNK),
            in_specs=[
                pl.BlockSpec(memory_space=pl.ANY),
                pl.BlockSpec((VPAD, DP), lambda p, c, toks: (0, 0)),
                pl.BlockSpec((1, DP), lambda p, c, toks: (0, 0)),
            ],
            out_specs=[
                pl.BlockSpec((1, 256, DP), lambda p, c, toks: (p, c, 0)),
                pl.BlockSpec((1, 2, DP), lambda p, c, toks: (p, 0, 0)),
            ],
            scratch_shapes=[
                pltpu.VMEM((2 * V, 128), jnp.int32),
                pltpu.VMEM((512, 128), jnp.int32),
                pltpu.SemaphoreType.DMA,
            ],
        ),
        compiler_params=pltpu.CompilerParams(
            dimension_semantics=("parallel", "arbitrary"),
            vmem_limit_bytes=56 * 1024 * 1024),
        name="embed_gather_proj",
    )(toks, tbl_i32, wproj, pbias)


def _sig(x):
    return 1.0 / (1.0 + jnp.exp(-x))


def _tanh(x):
    e = jnp.exp(-2.0 * x)
    return (1.0 - e) / (1.0 + e)


def _hilo(x):
    hi = x.astype(jnp.bfloat16)
    lo = (x - hi.astype(jnp.float32)).astype(jnp.bfloat16)
    return hi, lo


def _dot3(xhi, xlo, whi, wlo):
    return (jnp.dot(xhi, whi, preferred_element_type=jnp.float32)
            + jnp.dot(xhi, wlo, preferred_element_type=jnp.float32)
            + jnp.dot(xlo, whi, preferred_element_type=jnp.float32))


def _lstm_kernel(y_hbm, st_ref, bng_ref, bnb_ref,
                 wih_ref, whh_ref, bi_ref, bh_ref, hs_hbm,
                 xbuf, h_ref, c_ref, hout, sem_in, sem_out):
    p = pl.program_id(0)
    bsum = bi_ref[...] + bh_ref[...]
    n = jnp.float32(S * B)
    mu = st_ref[0, 0:1, :] / n
    var = st_ref[0, 1:2, :] / n - mu * mu
    rs = jax.lax.rsqrt(var + 1e-5)
    bng = bng_ref[...]
    bnb = bnb_ref[...]
    h_ref[...] = jnp.zeros_like(h_ref)
    c_ref[...] = jnp.zeros_like(c_ref)
    wih = wih_ref[...]
    whh = whh_ref[...]

    pltpu.make_async_copy(y_hbm.at[p, pl.ds(0, B)], xbuf.at[0],
                          sem_in.at[0]).start()

    def step(s, carry):
        slot = jax.lax.rem(s, 2)
        nslot = jax.lax.rem(s + 1, 2)

        @pl.when(s + 1 < S)
        def _():
            pltpu.make_async_copy(y_hbm.at[p, pl.ds((s + 1) * B, B)],
                                  xbuf.at[nslot], sem_in.at[nslot]).start()

        pltpu.make_async_copy(y_hbm.at[p, pl.ds(s * B, B)], xbuf.at[slot],
                              sem_in.at[slot]).wait()
        x = ((((xbuf[slot] - mu) * rs) * bng) + bnb).astype(jnp.bfloat16)
        hb = h_ref[...].astype(jnp.bfloat16)
        z = (jnp.dot(x, wih, preferred_element_type=jnp.float32)
             + jnp.dot(hb, whh, preferred_element_type=jnp.float32)
             + bsum)
        zi = z[:, 0 * H:1 * H]
        zf = z[:, 1 * H:2 * H]
        zg = z[:, 2 * H:3 * H]
        zo = z[:, 3 * H:4 * H]
        c = _sig(zf) * c_ref[...] + _sig(zi) * _tanh(zg)
        h = _sig(zo) * _tanh(c)
        c_ref[...] = c
        h_ref[...] = h

        @pl.when(s >= 2)
        def _():
            pltpu.make_async_copy(hout.at[slot], hs_hbm.at[p, s - 2],
                                  sem_out.at[slot]).wait()

        hout[slot] = h
        pltpu.make_async_copy(hout.at[slot], hs_hbm.at[p, s],
                              sem_out.at[slot]).start()
        return carry

    jax.lax.fori_loop(0, S, step, 0)
    pltpu.make_async_copy(hout.at[S % 2], hs_hbm.at[p, S - 2],
                          sem_out.at[S % 2]).wait()
    pltpu.make_async_copy(hout.at[(S - 1) % 2], hs_hbm.at[p, S - 1],
                          sem_out.at[(S - 1) % 2]).wait()


def _run_lstm(y_all, stats, bng, bnb, wih_t, whh_t, bih, bhh):
    return pl.pallas_call(
        _lstm_kernel,
        out_shape=jax.ShapeDtypeStruct((2, S, B, H), jnp.float32),
        grid=(2,),
        in_specs=[
            pl.BlockSpec(memory_space=pl.ANY),
            pl.BlockSpec((1, 2, DP), lambda p: (p, 0, 0)),
            pl.BlockSpec((1, DP), lambda p: (0, 0)),
            pl.BlockSpec((1, DP), lambda p: (0, 0)),
            pl.BlockSpec((DP, H4), lambda p: (0, 0)),
            pl.BlockSpec((H, H4), lambda p: (0, 0)),
            pl.BlockSpec((1, H4), lambda p: (0, 0)),
            pl.BlockSpec((1, H4), lambda p: (0, 0)),
        ],
        out_specs=pl.BlockSpec(memory_space=pl.ANY),
        scratch_shapes=[
            pltpu.VMEM((2, B, DP), jnp.float32),
            pltpu.VMEM((B, H), jnp.float32),
            pltpu.VMEM((B, H), jnp.float32),
            pltpu.VMEM((2, B, H), jnp.float32),
            pltpu.SemaphoreType.DMA((2,)),
            pltpu.SemaphoreType.DMA((2,)),
        ],
        compiler_params=pltpu.CompilerParams(
            dimension_semantics=("parallel",),
            vmem_limit_bytes=56 * 1024 * 1024),
        name="lstm_encoder",
    )(y_all, stats, bng, bnb, wih_t, whh_t, bih, bhh)


def _dist_kernel(h1_ref, h2_ref, o_ref):
    jj = jax.lax.broadcasted_iota(jnp.int32, (S, G), 1)
    j = jax.lax.broadcasted_iota(jnp.int32, (S, G), 0)
    colsel = (j == KP * jj).astype(jnp.bfloat16)
    ii = jax.lax.broadcasted_iota(jnp.int32, (G, S), 0)
    i2 = jax.lax.broadcasted_iota(jnp.int32, (G, S), 1)
    rowsel = (i2 == KP * ii).astype(jnp.bfloat16)
    ones_row = jnp.ones((1, H), jnp.bfloat16)

    for bi in range(BBLK):
        h1 = h1_ref[0, :, bi * H:(bi + 1) * H]
        h2 = h2_ref[0, :, bi * H:(bi + 1) * H]
        dg = lambda a, b: jax.lax.dot_general(
            a, b, (((1,), (1,)), ((), ())),
            preferred_element_type=jnp.float32)
        g = dg(h1.astype(jnp.bfloat16), h2.astype(jnp.bfloat16))
        n1 = jnp.sum(h1 * h1, axis=1, keepdims=True)
        sqhi, sqlo = _hilo(h2 * h2)
        n2 = dg(ones_row, sqhi) + dg(ones_row, sqlo)
        sq = n1 + n2 - 2.0 * g
        m = sq
        for k in (1, 2, 3):
            m = jnp.minimum(m, jnp.concatenate([sq[:, k:], sq[:, :k]], axis=1))
        mhi, mlo = _hilo(m)
        mc = (jnp.dot(mhi, colsel, preferred_element_type=jnp.float32)
              + jnp.dot(mlo, colsel, preferred_element_type=jnp.float32))
        m2 = mc
        for k in (1, 2, 3):
            m2 = jnp.minimum(m2, jnp.concatenate([mc[k:], mc[:k]], axis=0))
        m2hi, m2lo = _hilo(m2)
        pooled_sq = (jnp.dot(rowsel, m2hi, preferred_element_type=jnp.float32)
                     + jnp.dot(rowsel, m2lo, preferred_element_type=jnp.float32))
        o_ref[bi] = jnp.sqrt(jnp.maximum(pooled_sq, 0.0) + 1e-12)


def _run_dist(hs_flat):
    return pl.pallas_call(
        _dist_kernel,
        out_shape=jax.ShapeDtypeStruct((B, G, G), jnp.float32),
        grid=(B // BBLK,),
        in_specs=[
            pl.BlockSpec((1, S, BBLK * H), lambda i: (0, 0, i)),
            pl.BlockSpec((1, S, BBLK * H), lambda i: (1, 0, i)),
        ],
        out_specs=pl.BlockSpec((BBLK, G, G), lambda i: (i, 0, 0)),
        compiler_params=pltpu.CompilerParams(
            dimension_semantics=("parallel",),
            vmem_limit_bytes=56 * 1024 * 1024),
        name="pair_dist_pool",
    )(hs_flat, hs_flat)


def _bn_cols(x, g, b):
    mu = x.mean(axis=0, keepdims=True)
    var = x.var(axis=0, keepdims=True)
    return (x - mu) * jax.lax.rsqrt(var + 1e-5) * g + b


def _mlp_kernel(x_ref, w1_ref, b1_ref, g1_ref, be1_ref,
                w2_ref, b2_ref, g2_ref, be2_ref, w3_ref, b3_ref, o_ref):
    x = x_ref[...]
    z1 = jnp.dot(x, w1_ref[...], preferred_element_type=jnp.float32) + b1_ref[...]
    x1 = _bn_cols(jnp.maximum(z1, 0.0), g1_ref[...], be1_ref[...])
    z2 = jnp.dot(x1, w2_ref[...], preferred_element_type=jnp.float32) + b2_ref[...]
    x2 = _bn_cols(jnp.maximum(z2, 0.0), g2_ref[...], be2_ref[...])
    z3 = jnp.dot(x2, w3_ref[...], preferred_element_type=jnp.float32) + b3_ref[...]
    lane = jax.lax.broadcasted_iota(jnp.int32, z3.shape, 1)
    zm = jnp.where(lane < DOUT, z3, -jnp.inf)
    mx = jnp.max(zm, axis=-1, keepdims=True)
    lse = jnp.log(jnp.sum(jnp.where(lane < DOUT, jnp.exp(zm - mx), 0.0),
                          axis=-1, keepdims=True)) + mx
    o_ref[...] = z3 - lse


def _run_mlp(x, W1, b1, g1, be1, W2, b2, g2, be2, W3, b3):
    return pl.pallas_call(
        _mlp_kernel,
        out_shape=jax.ShapeDtypeStruct((B, 128), jnp.float32),
        name="mlp_head",
        compiler_params=pltpu.CompilerParams(
            vmem_limit_bytes=56 * 1024 * 1024),
    )(x, W1, b1, g1, be1, W2, b2, g2, be2, W3, b3)


def kernel(embed_table, proj_W, proj_b, bn_g, bn_b, Wih, Whh, bih, bhh,
           W1, b1, g1, be1, W2, b2, g2, be2, W3, b3,
           extra_feats, sentence1, sentence2):
    tbl_i32 = jax.lax.bitcast_convert_type(
        _run_pack(embed_table), jnp.int32).reshape(2 * V, 128)
    toks = jnp.concatenate([sentence1.reshape(-1),
                            sentence2.reshape(-1)]).astype(jnp.int32) * 2
    wproj = jnp.pad(proj_W.T, ((0, VPAD - D), (0, DP - D))
                    ).astype(jnp.bfloat16)
    pbias = jnp.pad(proj_b, (0, DP - D))[None, :]
    y_all, stats = _run_embed(toks, tbl_i32, wproj, pbias)

    wih_t = jnp.pad(Wih.T, ((0, DP - D), (0, 0))).astype(jnp.bfloat16)
    whh_t = Whh.T.astype(jnp.bfloat16)
    bng = jnp.pad(bn_g, (0, DP - D))[None, :]
    bnb = jnp.pad(bn_b, (0, DP - D))[None, :]
    hs = _run_lstm(y_all, stats, bng, bnb, wih_t, whh_t,
                   bih[None, :], bhh[None, :])

    pooled = _run_dist(hs.reshape(2, S, B * H))

    x = jnp.concatenate([pooled.reshape(B, G * G), extra_feats,
                         jnp.zeros((B, 256 - (G * G + 7)), jnp.float32)],
                        axis=1)
    W1p = jnp.pad(W1.T, ((0, 256 - (G * G + 7)), (0, 0)))
    W3p = jnp.pad(W3.T, ((0, 0), (0, 128 - DOUT)))
    out = _run_mlp(x, W1p, b1[None, :], g1[None, :], be1[None, :],
                   W2.T, b2[None, :], g2[None, :], be2[None, :], W3p,
                   jnp.pad(b3, (0, 126))[None, :])
    return out[:, :DOUT]


# speedup vs baseline: 1.6542x; 1.0304x over previous
import jax
import jax.numpy as jnp
from jax.experimental import pallas as pl
from jax.experimental.pallas import tpu as pltpu

S, B, V, D, H, G = 60, 256, 50000, 300, 512, 15
DM, DOUT = 1024, 2
KP = S // G
DP = 384
H4 = 4 * H
BBLK = 16


NCHUNK = S * B // 256
VPAD = 512


VROWS = 5000


def _pack_kernel(x_ref, o_ref):
    x = x_ref[...]
    a1 = jnp.concatenate(
        [x[:, 256:D], jnp.zeros((VROWS, 128 - (D - 256)), jnp.float32)],
        axis=1)
    zero = jnp.zeros((VROWS, 128), jnp.float32)
    w0 = pltpu.pack_elementwise([x[:, 0:128], x[:, 128:256]],
                                packed_dtype=jnp.bfloat16)
    w1 = pltpu.pack_elementwise([a1, zero], packed_dtype=jnp.bfloat16)
    o_ref[:, 0, :] = w0
    o_ref[:, 1, :] = w1


def _run_pack(embed_table):
    return pl.pallas_call(
        _pack_kernel,
        out_shape=jax.ShapeDtypeStruct((V, 2, 128), jnp.uint32),
        grid=(V // VROWS,),
        in_specs=[pl.BlockSpec((VROWS, D), lambda i: (i, 0))],
        out_specs=pl.BlockSpec((VROWS, 2, 128), lambda i: (i, 0, 0)),
        compiler_params=pltpu.CompilerParams(
            dimension_semantics=("arbitrary",),
            vmem_limit_bytes=56 * 1024 * 1024),
        name="table_pack",
    )(embed_table)


def _embed_kernel(toks_ref, tbl_hbm, wp_ref, pb_ref, y_ref, st_ref,
                  tbl_vmem, tile, sem):
    p = pl.program_id(0)
    c = pl.program_id(1)

    @pl.when(c == 0)
    def _():
        cp = pltpu.make_async_copy(tbl_hbm, tbl_vmem, sem)
        cp.start()
        cp.wait()

    off = p * (S * B) + c * 256
    for mi in range(256):
        t = pl.multiple_of(toks_ref[off + mi], 2)
        tile[2 * mi:2 * mi + 2, :] = tbl_vmem[pl.ds(t, 2), :]

    tv = pltpu.bitcast(tile[...], jnp.bfloat16)
    e4 = tv.reshape(256, 4, 128)
    y = pb_ref[...].astype(jnp.float32)
    acc = None
    for c4 in range(4):
        d = jnp.dot(e4[:, c4, :], wp_ref[128 * c4:128 * (c4 + 1), :],
                    preferred_element_type=jnp.float32)
        acc = d if acc is None else acc + d
    y = acc + y
    y_ref[0] = y

    @pl.when(c == 0)
    def _():
        st_ref[...] = jnp.zeros_like(st_ref)

    st_ref[0, 0:1, :] = st_ref[0, 0:1, :] + jnp.sum(y, axis=0, keepdims=True)
    st_ref[0, 1:2, :] = st_ref[0, 1:2, :] + jnp.sum(y * y, axis=0,
                                                    keepdims=True)


def _run_embed(toks, tbl_i32, wproj, pbias):
    return pl.pallas_call(
        _embed_kernel,
        out_shape=(jax.ShapeDtypeStruct((2, S * B, DP), jnp.float32),
                   jax.ShapeDtypeStruct((2, 2, DP), jnp.float32)),
        grid_spec=pltpu.PrefetchScalarGridSpec(
            num_scalar_prefetch=1,
            grid=(2, NCHUNK),
            in_specs=[
                pl.BlockSpec(memory_space=pl.ANY),
                pl.BlockSpec((VPAD, DP), lambda p, c, toks: (0, 0)),
                pl.BlockSpec((1, DP), lambda p, c, toks: (0, 0)),
            ],
            out_specs=[
                pl.BlockSpec((1, 256, DP), lambda p, c, toks: (p, c, 0)),
                pl.BlockSpec((1, 2, DP), lambda p, c, toks: (p, 0, 0)),
            ],
            scratch_shapes=[
                pltpu.VMEM((2 * V, 128), jnp.int32),
                pltpu.VMEM((512, 128), jnp.int32),
                pltpu.SemaphoreType.DMA,
            ],
        ),
        compiler_params=pltpu.CompilerParams(
            dimension_semantics=("parallel", "arbitrary"),
            vmem_limit_bytes=56 * 1024 * 1024),
        name="embed_gather_proj",
    )(toks, tbl_i32, wproj, pbias)


def _hilo(x):
    hi = x.astype(jnp.bfloat16)
    lo = (x - hi.astype(jnp.float32)).astype(jnp.bfloat16)
    return hi, lo


def _lstm_kernel(y_hbm, st_ref, bng_ref, bnb_ref,
                 wih_ref, whh_ref, bi_ref, bh_ref, hs_hbm,
                 xbuf, h_ref, c_ref, hout, sem_in, sem_out):
    p = pl.program_id(0)
    bsum = bi_ref[...] + bh_ref[...]
    n = jnp.float32(S * B)
    mu = st_ref[0, 0:1, :] / n
    var = st_ref[0, 1:2, :] / n - mu * mu
    rs = jax.lax.rsqrt(var + 1e-5)
    bng = bng_ref[...]
    bnb = bnb_ref[...]
    h_ref[...] = jnp.zeros_like(h_ref)
    c_ref[...] = jnp.zeros_like(c_ref)
    wih = wih_ref[...]
    whh = whh_ref[...]

    pltpu.make_async_copy(y_hbm.at[p, pl.ds(0, B)], xbuf.at[0],
                          sem_in.at[0]).start()

    def step(s, carry):
        slot = jax.lax.rem(s, 2)
        nslot = jax.lax.rem(s + 1, 2)

        @pl.when(s + 1 < S)
        def _():
            pltpu.make_async_copy(y_hbm.at[p, pl.ds((s + 1) * B, B)],
                                  xbuf.at[nslot], sem_in.at[nslot]).start()

        pltpu.make_async_copy(y_hbm.at[p, pl.ds(s * B, B)], xbuf.at[slot],
                              sem_in.at[slot]).wait()
        x = ((((xbuf[slot] - mu) * rs) * bng) + bnb).astype(jnp.bfloat16)
        hb = h_ref[...].astype(jnp.bfloat16)
        z = (jnp.dot(x, wih, preferred_element_type=jnp.float32)
             + jnp.dot(hb, whh, preferred_element_type=jnp.float32)
             + bsum)
        zi = z[:, 0 * H:1 * H]
        zf = z[:, 1 * H:2 * H]
        zg = z[:, 2 * H:3 * H]
        zo = z[:, 3 * H:4 * H]
        c = (jax.nn.sigmoid(zf) * c_ref[...]
             + jax.nn.sigmoid(zi) * jnp.tanh(zg))
        h = jax.nn.sigmoid(zo) * jnp.tanh(c)
        c_ref[...] = c
        h_ref[...] = h

        @pl.when(s >= 2)
        def _():
            pltpu.make_async_copy(hout.at[slot], hs_hbm.at[p, s - 2],
                                  sem_out.at[slot]).wait()

        hout[slot] = h
        pltpu.make_async_copy(hout.at[slot], hs_hbm.at[p, s],
                              sem_out.at[slot]).start()
        return carry

    jax.lax.fori_loop(0, S, step, 0)
    pltpu.make_async_copy(hout.at[S % 2], hs_hbm.at[p, S - 2],
                          sem_out.at[S % 2]).wait()
    pltpu.make_async_copy(hout.at[(S - 1) % 2], hs_hbm.at[p, S - 1],
                          sem_out.at[(S - 1) % 2]).wait()


def _run_lstm(y_all, stats, bng, bnb, wih_t, whh_t, bih, bhh):
    return pl.pallas_call(
        _lstm_kernel,
        out_shape=jax.ShapeDtypeStruct((2, S, B, H), jnp.float32),
        grid=(2,),
        in_specs=[
            pl.BlockSpec(memory_space=pl.ANY),
            pl.BlockSpec((1, 2, DP), lambda p: (p, 0, 0)),
            pl.BlockSpec((1, DP), lambda p: (0, 0)),
            pl.BlockSpec((1, DP), lambda p: (0, 0)),
            pl.BlockSpec((DP, H4), lambda p: (0, 0)),
            pl.BlockSpec((H, H4), lambda p: (0, 0)),
            pl.BlockSpec((1, H4), lambda p: (0, 0)),
            pl.BlockSpec((1, H4), lambda p: (0, 0)),
        ],
        out_specs=pl.BlockSpec(memory_space=pl.ANY),
        scratch_shapes=[
            pltpu.VMEM((2, B, DP), jnp.float32),
            pltpu.VMEM((B, H), jnp.float32),
            pltpu.VMEM((B, H), jnp.float32),
            pltpu.VMEM((2, B, H), jnp.float32),
            pltpu.SemaphoreType.DMA((2,)),
            pltpu.SemaphoreType.DMA((2,)),
        ],
        compiler_params=pltpu.CompilerParams(
            dimension_semantics=("parallel",),
            vmem_limit_bytes=56 * 1024 * 1024),
        name="lstm_encoder",
    )(y_all, stats, bng, bnb, wih_t, whh_t, bih, bhh)


def _dist_kernel(h1_ref, h2_ref, o_ref):
    jj = jax.lax.broadcasted_iota(jnp.int32, (S, G), 1)
    j = jax.lax.broadcasted_iota(jnp.int32, (S, G), 0)
    colsel = (j == KP * jj).astype(jnp.bfloat16)
    ii = jax.lax.broadcasted_iota(jnp.int32, (G, S), 0)
    i2 = jax.lax.broadcasted_iota(jnp.int32, (G, S), 1)
    rowsel = (i2 == KP * ii).astype(jnp.bfloat16)
    ones_row = jnp.ones((1, H), jnp.bfloat16)

    for bi in range(BBLK):
        h1 = h1_ref[0, :, bi * H:(bi + 1) * H]
        h2 = h2_ref[0, :, bi * H:(bi + 1) * H]
        dg = lambda a, b: jax.lax.dot_general(
            a, b, (((1,), (1,)), ((), ())),
            preferred_element_type=jnp.float32)
        g = dg(h1.astype(jnp.bfloat16), h2.astype(jnp.bfloat16))
        n1 = jnp.sum(h1 * h1, axis=1, keepdims=True)
        sqhi, sqlo = _hilo(h2 * h2)
        n2 = dg(ones_row, sqhi) + dg(ones_row, sqlo)
        sq = n1 + n2 - 2.0 * g
        m = sq
        for k in (1, 2, 3):
            m = jnp.minimum(m, jnp.concatenate([sq[:, k:], sq[:, :k]], axis=1))
        mhi, mlo = _hilo(m)
        mc = (jnp.dot(mhi, colsel, preferred_element_type=jnp.float32)
              + jnp.dot(mlo, colsel, preferred_element_type=jnp.float32))
        m2 = mc
        for k in (1, 2, 3):
            m2 = jnp.minimum(m2, jnp.concatenate([mc[k:], mc[:k]], axis=0))
        m2hi, m2lo = _hilo(m2)
        pooled_sq = (jnp.dot(rowsel, m2hi, preferred_element_type=jnp.float32)
                     + jnp.dot(rowsel, m2lo, preferred_element_type=jnp.float32))
        o_ref[bi] = jnp.sqrt(jnp.maximum(pooled_sq, 0.0) + 1e-12)


def _run_dist(hs_flat):
    return pl.pallas_call(
        _dist_kernel,
        out_shape=jax.ShapeDtypeStruct((B, G, G), jnp.float32),
        grid=(B // BBLK,),
        in_specs=[
            pl.BlockSpec((1, S, BBLK * H), lambda i: (0, 0, i)),
            pl.BlockSpec((1, S, BBLK * H), lambda i: (1, 0, i)),
        ],
        out_specs=pl.BlockSpec((BBLK, G, G), lambda i: (i, 0, 0)),
        compiler_params=pltpu.CompilerParams(
            dimension_semantics=("parallel",),
            vmem_limit_bytes=56 * 1024 * 1024),
        name="pair_dist_pool",
    )(hs_flat, hs_flat)


def _bn_cols(x, g, b):
    mu = x.mean(axis=0, keepdims=True)
    var = x.var(axis=0, keepdims=True)
    return (x - mu) * jax.lax.rsqrt(var + 1e-5) * g + b


def _mlp_kernel(x_ref, w1_ref, b1_ref, g1_ref, be1_ref,
                w2_ref, b2_ref, g2_ref, be2_ref, w3_ref, b3_ref, o_ref):
    x = x_ref[...]
    z1 = jnp.dot(x, w1_ref[...], preferred_element_type=jnp.float32) + b1_ref[...]
    x1 = _bn_cols(jnp.maximum(z1, 0.0), g1_ref[...], be1_ref[...])
    z2 = jnp.dot(x1, w2_ref[...], preferred_element_type=jnp.float32) + b2_ref[...]
    x2 = _bn_cols(jnp.maximum(z2, 0.0), g2_ref[...], be2_ref[...])
    z3 = jnp.dot(x2, w3_ref[...], preferred_element_type=jnp.float32) + b3_ref[...]
    lane = jax.lax.broadcasted_iota(jnp.int32, z3.shape, 1)
    zm = jnp.where(lane < DOUT, z3, -jnp.inf)
    mx = jnp.max(zm, axis=-1, keepdims=True)
    lse = jnp.log(jnp.sum(jnp.where(lane < DOUT, jnp.exp(zm - mx), 0.0),
                          axis=-1, keepdims=True)) + mx
    o_ref[...] = z3 - lse


def _run_mlp(x, W1, b1, g1, be1, W2, b2, g2, be2, W3, b3):
    return pl.pallas_call(
        _mlp_kernel,
        out_shape=jax.ShapeDtypeStruct((B, 128), jnp.float32),
        name="mlp_head",
        compiler_params=pltpu.CompilerParams(
            vmem_limit_bytes=56 * 1024 * 1024),
    )(x, W1, b1, g1, be1, W2, b2, g2, be2, W3, b3)


def kernel(embed_table, proj_W, proj_b, bn_g, bn_b, Wih, Whh, bih, bhh,
           W1, b1, g1, be1, W2, b2, g2, be2, W3, b3,
           extra_feats, sentence1, sentence2):
    tbl_i32 = jax.lax.bitcast_convert_type(
        _run_pack(embed_table), jnp.int32).reshape(2 * V, 128)
    toks = jnp.concatenate([sentence1.reshape(-1),
                            sentence2.reshape(-1)]).astype(jnp.int32) * 2
    wproj = jnp.pad(proj_W.T, ((0, VPAD - D), (0, DP - D))
                    ).astype(jnp.bfloat16)
    pbias = jnp.pad(proj_b, (0, DP - D))[None, :]
    y_all, stats = _run_embed(toks, tbl_i32, wproj, pbias)

    wih_t = jnp.pad(Wih.T, ((0, DP - D), (0, 0))).astype(jnp.bfloat16)
    whh_t = Whh.T.astype(jnp.bfloat16)
    bng = jnp.pad(bn_g, (0, DP - D))[None, :]
    bnb = jnp.pad(bn_b, (0, DP - D))[None, :]
    hs = _run_lstm(y_all, stats, bng, bnb, wih_t, whh_t,
                   bih[None, :], bhh[None, :])

    pooled = _run_dist(hs.reshape(2, S, B * H))

    x = jnp.concatenate([pooled.reshape(B, G * G), extra_feats,
                         jnp.zeros((B, 256 - (G * G + 7)), jnp.float32)],
                        axis=1)
    W1p = jnp.pad(W1.T, ((0, 256 - (G * G + 7)), (0, 0)))
    W3p = jnp.pad(W3.T, ((0, 0), (0, 128 - DOUT)))
    out = _run_mlp(x, W1p, b1[None, :], g1[None, :], be1[None, :],
                   W2.T, b2[None, :], g2[None, :], be2[None, :], W3p,
                   jnp.pad(b3, (0, 126))[None, :])
    return out[:, :DOUT]
